# Initial kernel scaffold; baseline (speedup 1.0000x reference)
import jax
import jax.numpy as jnp
from jax.experimental import pallas as pl


def kernel(in_vec):
    raise NotImplementedError("write your pallas kernel here")


# speedup vs baseline: 2.5213x; 2.5213x over previous
import functools

import jax
import jax.numpy as jnp
from jax import lax
from jax.experimental import pallas as pl
from jax.experimental.pallas import tpu as pltpu
from jax.experimental.pallas import tpu_sc as plsc

N = 16384
CHAMPS = 10
SLOTS = 6
ITEMS1 = 201
WORDS = CHAMPS * ITEMS1 * N
NC, NS, L = 2, 16, 16
NW = NC * NS
WORDS_PER_W = WORDS // NW
CHUNK_W = 16080
CHUNKS_PER_W = WORDS_PER_W // CHUNK_W
NCHUNK = NW * CHUNKS_PER_W
SUPER = 8
NUPD = N * CHAMPS * SLOTS
BLK = 768
GROUPS = BLK // L
PAD = BLK + 8
SB = 80


def _sc_scatter_sorted(idx_pad, val_pad, keep_pad, starts_pad):
    mesh = plsc.VectorSubcoreMesh(
        core_axis_name="c", subcore_axis_name="s",
        num_cores=NC, num_subcores=NS)

    @functools.partial(
        pl.kernel, mesh=mesh,
        compiler_params=pltpu.CompilerParams(needs_layout_passes=False),
        out_type=jax.ShapeDtypeStruct((WORDS,), jnp.float32),
        scratch_types=[
            pltpu.VMEM((BLK,), jnp.int32),
            pltpu.VMEM((BLK,), jnp.float32),
            pltpu.VMEM((BLK,), jnp.int32),
            pltpu.VMEM((SB,), jnp.int32),
            pltpu.VMEM((CHUNK_W,), jnp.float32),
        ],
    )
    def k(idx_hbm, val_hbm, keep_hbm, starts_hbm, out_hbm, idx_buf, val_buf,
          keep_buf, sbuf, out_buf):
        wid = lax.axis_index("s") * NC + lax.axis_index("c")
        pltpu.sync_copy(starts_hbm.at[pl.ds(wid * CHUNKS_PER_W, SB)], sbuf)
        zero16 = jnp.zeros((L,), jnp.float32)

        def zbody(i, carry):
            out_buf[pl.ds(i * L, L)] = zero16
            return carry
        lax.fori_loop(0, CHUNK_W // L, zbody, 0)

        def super_body(sc, carry):
            bvec = sbuf[pl.ds(sc * SUPER, 2 * SUPER)]
            for j in range(SUPER):
                u0 = bvec[j]
                u1 = bvec[j + 1]
                base0 = (u0 // 8) * 8
                word0 = (wid * CHUNKS_PER_W + sc * SUPER + j) * CHUNK_W
                nb = (u1 - base0 + (BLK - 1)) // BLK

                def blk_body(bi, c2):
                    ub = base0 + bi * BLK
                    pltpu.sync_copy(idx_hbm.at[pl.ds(ub, BLK)], idx_buf)
                    pltpu.sync_copy(val_hbm.at[pl.ds(ub, BLK)], val_buf)
                    pltpu.sync_copy(keep_hbm.at[pl.ds(ub, BLK)], keep_buf)
                    for g in range(GROUPS):
                        idx16 = idx_buf[pl.ds(g * L, L)]
                        val16 = val_buf[pl.ds(g * L, L)]
                        kp16 = keep_buf[pl.ds(g * L, L)]
                        loc = idx16 - word0
                        m = ((kp16 != 0) & (loc >= 0) & (loc < CHUNK_W))
                        plsc.store_scatter(out_buf, [loc], val16, mask=m)
                    return c2
                lax.fori_loop(0, nb, blk_body, 0)

                pltpu.sync_copy(out_buf, out_hbm.at[pl.ds(word0, CHUNK_W)])

                def blkz_body(bi, c2):
                    ub = base0 + bi * BLK
                    pltpu.sync_copy(idx_hbm.at[pl.ds(ub, BLK)], idx_buf)
                    for g in range(GROUPS):
                        idx16 = idx_buf[pl.ds(g * L, L)]
                        loc = idx16 - word0
                        m = (loc >= 0) & (loc < CHUNK_W)
                        plsc.store_scatter(out_buf, [loc], zero16, mask=m)
                    return c2
                lax.fori_loop(0, nb, blkz_body, 0)
            return carry
        lax.fori_loop(0, CHUNKS_PER_W // SUPER, super_body, 0)

    return k(idx_pad, val_pad, keep_pad, starts_pad)


def kernel(in_vec):
    items = in_vec[:, 11:131].reshape(N, CHAMPS, SLOTS, 2)
    ids = (items[..., 0] + 1.0).astype(jnp.int32)
    vals = items[..., 1].reshape(-1)
    ids = jnp.where(ids < 0, ids + ITEMS1, ids)
    inb = (ids >= 0) & (ids < ITEMS1)
    b_idx = lax.broadcasted_iota(jnp.int32, (N, CHAMPS, SLOTS), 0)
    c_idx = lax.broadcasted_iota(jnp.int32, (N, CHAMPS, SLOTS), 1)
    flat = (c_idx * ITEMS1 + ids) * N + b_idx
    flat = jnp.where(inb, flat, -1).reshape(-1)
    s_idx, s_val = lax.sort_key_val(flat, vals, is_stable=False)
    nxt = jnp.concatenate([s_idx[1:], jnp.full((1,), -2, jnp.int32)])
    keep = (s_idx != nxt).astype(jnp.int32)
    bounds = jnp.arange(NCHUNK + 1, dtype=jnp.int32) * CHUNK_W
    starts = jnp.searchsorted(s_idx, bounds).astype(jnp.int32)
    starts_pad = jnp.concatenate(
        [starts, jnp.full((SB,), NUPD, jnp.int32)])
    idx_pad = jnp.concatenate([s_idx, jnp.full((PAD,), -2, jnp.int32)])
    val_pad = jnp.concatenate([s_val, jnp.zeros((PAD,), jnp.float32)])
    keep_pad = jnp.concatenate([keep, jnp.zeros((PAD,), jnp.int32)])
    out = _sc_scatter_sorted(idx_pad, val_pad, keep_pad, starts_pad)
    return out.reshape(CHAMPS, ITEMS1, N).transpose(2, 0, 1)
